# Initial kernel scaffold; baseline (speedup 1.0000x reference)
import jax
import jax.numpy as jnp
from jax.experimental import pallas as pl


def kernel(x, edge_index, W1, U1, C1, B1, W2, U2, C2, B2, W3, U3, C3, B3, W4, U4, C4, B4, W5, U5, C5, B5, W6, U6, C6, B6, L1W, L1B, L2W, L2B, LOW, LOB):
    raise NotImplementedError("write your pallas kernel here")


# speedup vs baseline: 15.8338x; 15.8338x over previous
import functools

import jax
import jax.numpy as jnp
from jax import lax
from jax.experimental import pallas as pl
from jax.experimental.pallas import tpu as pltpu
from jax.experimental.pallas import tpu_sc as plsc

N = 10000
HEADS = 4
NC, NS, L = 2, 16, 16
NW = NC * NS
KB = 128
NPAD = 10240
RSTR = NPAD // NS

_SC_PARAMS = pltpu.CompilerParams(
    needs_layout_passes=False, use_tc_tiling_on_sc=False)


def _ceil_div(a, b):
    return -(-a // b)


@functools.cache
def _edge_a(nblk, with_count):
    cols = 32 if with_count else 16
    mesh = plsc.VectorSubcoreMesh(
        core_axis_name="c", subcore_axis_name="s", num_cores=NC, num_subcores=NS)

    @functools.partial(
        pl.kernel,
        out_type=jax.ShapeDtypeStruct((NC, NPAD, cols), jnp.float32),
        mesh=mesh,
        scratch_types=[
            pltpu.VMEM((nblk, KB), jnp.int32),
            pltpu.VMEM((nblk, KB), jnp.int32),
            pltpu.VMEM((N * HEADS,), jnp.float32),
            pltpu.VMEM((HEADS, L), jnp.float32),
            pltpu.VMEM((KB, 64), jnp.float32),
            pltpu.VMEM((KB, cols), jnp.float32),
            pltpu.VMEM((RSTR, cols), jnp.float32),
            pltpu.VMEM_SHARED((NPAD, cols), jnp.float32),
        ],
        compiler_params=_SC_PARAMS,
    )
    def body(src_hbm, dst_hbm, xw_hbm, xu_hbm, cb_hbm, out_hbm,
             src_v, dst_v, xu_v, cb_v, rows_v, msg_v, zb_v, acc_sh):
        cid = lax.axis_index("c")
        sid = lax.axis_index("s")
        wid = cid * NS + sid

        zero16 = jnp.zeros((L,), jnp.float32)

        def zrow(i, _):
            for j in range(cols // L):
                zb_v[i, pl.ds(j * L, L)] = zero16
            return 0

        lax.fori_loop(0, RSTR, zrow, 0)
        pltpu.sync_copy(zb_v, acc_sh.at[pl.ds(sid * RSTR, RSTR)])

        pltpu.sync_copy(xu_hbm, xu_v)
        pltpu.sync_copy(cb_hbm, cb_v)
        pltpu.sync_copy(src_hbm.at[wid], src_v)
        pltpu.sync_copy(dst_hbm.at[wid], dst_v)

        if with_count:
            one0 = jnp.where(lax.iota(jnp.int32, L) == 0, 1.0, 0.0).astype(jnp.float32)

            def onerow(i, _):
                msg_v[i, pl.ds(L, L)] = one0
                return 0

            lax.fori_loop(0, KB, onerow, 0)

        plsc.subcore_barrier()

        def block(b, _):
            pltpu.sync_copy(xw_hbm.at[src_v.at[b]], rows_v)
            for g in range(KB // L):
                s16 = src_v[b, pl.ds(g * L, L)] * HEADS
                d16 = dst_v[b, pl.ds(g * L, L)] * HEADS
                zs = []
                for h in range(HEADS):
                    zh = (plsc.load_gather(xu_v, [s16 + h])
                          - plsc.load_gather(xu_v, [d16 + h])
                          + cb_v[h, :])
                    zs.append(zh)
                m = jnp.maximum(jnp.maximum(zs[0], zs[1]), jnp.maximum(zs[2], zs[3]))
                es = [jnp.exp(z - m) for z in zs]
                rinv = 1.0 / (es[0] + es[1] + es[2] + es[3])
                qs = [e * rinv for e in es]
                for e in range(L):
                    r = g * L + e
                    acc = qs[0][e] * rows_v[r, pl.ds(0, L)]
                    for h in range(1, HEADS):
                        acc = acc + qs[h][e] * rows_v[r, pl.ds(h * L, L)]
                    msg_v[r, pl.ds(0, L)] = acc
            pltpu.sync_copy(msg_v, acc_sh.at[dst_v.at[b]], add=True)
            return 0

        lax.fori_loop(0, nblk, block, 0)

        plsc.subcore_barrier()
        pltpu.sync_copy(acc_sh.at[pl.ds(sid * RSTR, RSTR)],
                        out_hbm.at[cid, pl.ds(sid * RSTR, RSTR)])

    return body


@functools.cache
def _edge_b(nblk, cout):
    mesh = plsc.VectorSubcoreMesh(
        core_axis_name="c", subcore_axis_name="s", num_cores=NC, num_subcores=NS)

    @functools.partial(
        pl.kernel,
        out_type=jax.ShapeDtypeStruct((NC, NPAD, cout), jnp.float32),
        mesh=mesh,
        scratch_types=[
            pltpu.VMEM((nblk, KB), jnp.int32),
            pltpu.VMEM((nblk, KB), jnp.int32),
            pltpu.VMEM((KB, cout), jnp.float32),
            pltpu.VMEM((RSTR, cout), jnp.float32),
            pltpu.VMEM_SHARED((NPAD, cout), jnp.float32),
        ],
        compiler_params=_SC_PARAMS,
    )
    def body(src_hbm, dst_hbm, xw_hbm, out_hbm,
             src_v, dst_v, rows_v, zb_v, acc_sh):
        cid = lax.axis_index("c")
        sid = lax.axis_index("s")
        wid = cid * NS + sid

        zero16 = jnp.zeros((L,), jnp.float32)

        def zrow(i, _):
            for j in range(cout // L):
                zb_v[i, pl.ds(j * L, L)] = zero16
            return 0

        lax.fori_loop(0, RSTR, zrow, 0)
        pltpu.sync_copy(zb_v, acc_sh.at[pl.ds(sid * RSTR, RSTR)])
        pltpu.sync_copy(src_hbm.at[wid], src_v)
        pltpu.sync_copy(dst_hbm.at[wid], dst_v)

        plsc.subcore_barrier()

        def block(b, _):
            pltpu.sync_copy(xw_hbm.at[src_v.at[b]], rows_v)
            pltpu.sync_copy(rows_v, acc_sh.at[dst_v.at[b]], add=True)
            return 0

        lax.fori_loop(0, nblk, block, 0)

        plsc.subcore_barrier()
        pltpu.sync_copy(acc_sh.at[pl.ds(sid * RSTR, RSTR)],
                        out_hbm.at[cid, pl.ds(sid * RSTR, RSTR)])

    return body


def _mm(a, b):
    return jnp.dot(a, b, preferred_element_type=jnp.float32)


def _prep1_body(x_ref, w_ref, u_ref, xw_ref, xu_ref):
    x = x_ref[...]
    xw_ref[...] = _mm(x, w_ref[...])
    xu_ref[...] = _mm(x, u_ref[...])


def _prep2_body(acc_ref, b_ref, w_ref, u_ref, x1_ref, xw_ref, xu_ref, inv_ref):
    cnt = acc_ref[0, :N, 16:17] + acc_ref[1, :N, 16:17]
    inv = 1.0 / cnt
    inv_ref[...] = inv
    xl = jax.nn.relu((acc_ref[0, :N, 0:16] + acc_ref[1, :N, 0:16]) * inv + b_ref[...])
    x1_ref[...] = xl
    xw_ref[...] = _mm(xl, w_ref[...])
    xu_ref[...] = _mm(xl, u_ref[...])


def _prep_a_body(acc_ref, inv_ref, b_ref, w_ref, u_ref, xw_ref, xu_ref):
    xl = jax.nn.relu((acc_ref[0, :N, :] + acc_ref[1, :N, :]) * inv_ref[...] + b_ref[...])
    xw_ref[...] = _mm(xl, w_ref[...])
    xu_ref[...] = _mm(xl, u_ref[...])


def _prep_b4_body(acc_ref, inv_ref, b_ref, w_ref, x2_ref, xw_ref):
    xl = jax.nn.relu((acc_ref[0, :N, :] + acc_ref[1, :N, :]) * inv_ref[...] + b_ref[...])
    x2_ref[...] = xl
    xw_ref[...] = _mm(xl, w_ref[...])


def _prep_b_body(acc_ref, inv_ref, b_ref, w_ref, xw_ref):
    xl = jax.nn.relu((acc_ref[0, :N, :] + acc_ref[1, :N, :]) * inv_ref[...] + b_ref[...])
    xw_ref[...] = _mm(xl, w_ref[...])


def _head_body(x1_ref, x2_ref, acc_ref, inv_ref, b6_ref,
               l1w_ref, l1b_ref, l2w_ref, l2b_ref, low_ref, lob_ref, out_ref):
    x3 = (acc_ref[0, :N, :] + acc_ref[1, :N, :]) * inv_ref[...] + b6_ref[...]
    x4 = (_mm(x1_ref[...], l1w_ref[0:16, :])
          + _mm(x2_ref[...], l1w_ref[16:32, :])
          + _mm(jax.nn.relu(x3), l1w_ref[32:96, :])
          + l1b_ref[...])
    x5 = _mm(jax.nn.relu(x4), l2w_ref[...]) + l2b_ref[...]
    x6 = _mm(jax.nn.relu(x5), low_ref[...]) + lob_ref[...]
    out_ref[...] = 1.0 / (1.0 + jnp.exp(-x6))


def _tc(body, out_shapes, *args):
    return pl.pallas_call(body, out_shape=out_shapes)(*args)


def kernel(x, edge_index, W1, U1, C1, B1, W2, U2, C2, B2, W3, U3, C3, B3,
           W4, U4, C4, B4, W5, U5, C5, B5, W6, U6, C6, B6,
           L1W, L1B, L2W, L2B, LOW, LOB):
    e_raw = edge_index.shape[1]
    e_tot = e_raw + N
    nblk = _ceil_div(e_tot, NW * KB)
    ep = NW * nblk * KB
    pad = ep - e_tot

    loop_idx = jnp.arange(N, dtype=jnp.int32)
    src = jnp.concatenate([
        edge_index[0].astype(jnp.int32), loop_idx,
        jnp.zeros((pad,), jnp.int32)]).reshape(NW, nblk, KB)
    dst = jnp.concatenate([
        edge_index[1].astype(jnp.int32), loop_idx,
        jnp.full((pad,), N, jnp.int32)]).reshape(NW, nblk, KB)

    f32 = jnp.float32
    sd = jax.ShapeDtypeStruct

    cb1 = jnp.broadcast_to(C1[:, None], (HEADS, L))
    cb2 = jnp.broadcast_to(C2[:, None], (HEADS, L))
    cb3 = jnp.broadcast_to(C3[:, None], (HEADS, L))

    xw1, xu1 = _tc(_prep1_body, (sd((N, 64), f32), sd((N, HEADS), f32)),
                   x, W1, U1)
    acc1 = _edge_a(nblk, True)(src, dst, xw1, xu1.reshape(-1), cb1)

    x1, xw2, xu2, inv = _tc(
        _prep2_body,
        (sd((N, 16), f32), sd((N, 64), f32), sd((N, HEADS), f32), sd((N, 1), f32)),
        acc1, B1, W2, U2)
    acc2 = _edge_a(nblk, False)(src, dst, xw2, xu2.reshape(-1), cb2)

    xw3, xu3 = _tc(_prep_a_body, (sd((N, 64), f32), sd((N, HEADS), f32)),
                   acc2, inv, B2, W3, U3)
    acc3 = _edge_a(nblk, False)(src, dst, xw3, xu3.reshape(-1), cb3)

    x2, xw4 = _tc(_prep_b4_body, (sd((N, 16), f32), sd((N, 16), f32)),
                  acc3, inv, B3, W4)
    acc4 = _edge_b(nblk, 16)(src, dst, xw4)

    xw5 = _tc(_prep_b_body, sd((N, 32), f32), acc4, inv, B4, W5)
    acc5 = _edge_b(nblk, 32)(src, dst, xw5)

    xw6 = _tc(_prep_b_body, sd((N, 64), f32), acc5, inv, B5, W6)
    acc6 = _edge_b(nblk, 64)(src, dst, xw6)

    out = _tc(_head_body, sd((N, 1), f32),
              x1, x2, acc6, inv, B6, L1W, L1B, L2W, L2B, LOW, LOB)
    return out


# speedup vs baseline: 21.2704x; 1.3434x over previous
import functools

import jax
import jax.numpy as jnp
from jax import lax
from jax.experimental import pallas as pl
from jax.experimental.pallas import tpu as pltpu
from jax.experimental.pallas import tpu_sc as plsc

N = 10000
HEADS = 4
NC, NS, L = 2, 16, 16
NW = NC * NS
KB = 128
NPAD = 10240
RSTR = NPAD // NS
NBUF = 3

_SC_PARAMS = pltpu.CompilerParams(
    needs_layout_passes=False, use_tc_tiling_on_sc=False)


def _ceil_div(a, b):
    return -(-a // b)


@functools.cache
def _edge_a(nblk, with_count):
    cols = 32 if with_count else 16
    mesh = plsc.VectorSubcoreMesh(
        core_axis_name="c", subcore_axis_name="s", num_cores=NC, num_subcores=NS)

    @functools.partial(
        pl.kernel,
        out_type=jax.ShapeDtypeStruct((NC, NPAD, cols), jnp.float32),
        mesh=mesh,
        scratch_types=[
            pltpu.VMEM((nblk, KB), jnp.int32),
            pltpu.VMEM((nblk, KB), jnp.int32),
            pltpu.VMEM((N * HEADS,), jnp.float32),
            pltpu.VMEM((HEADS, L), jnp.float32),
            pltpu.VMEM((NBUF, KB, 64), jnp.float32),
            pltpu.VMEM((KB, cols), jnp.float32),
            pltpu.VMEM((RSTR, cols), jnp.float32),
            pltpu.VMEM_SHARED((NPAD, cols), jnp.float32),
            pltpu.SemaphoreType.DMA,
            pltpu.SemaphoreType.DMA,
            pltpu.SemaphoreType.DMA,
        ],
        compiler_params=_SC_PARAMS,
    )
    def body(src_hbm, dst_hbm, xw_hbm, xu_hbm, cb_hbm, out_hbm,
             src_v, dst_v, xu_v, cb_v, rows_v, msg_v, zb_v, acc_sh,
             sem0, sem1, sem2):
        sems = (sem0, sem1, sem2)
        cid = lax.axis_index("c")
        sid = lax.axis_index("s")
        wid = cid * NS + sid

        zero16 = jnp.zeros((L,), jnp.float32)

        def zrow(i, _):
            for j in range(cols // L):
                zb_v[i, pl.ds(j * L, L)] = zero16
            return 0

        lax.fori_loop(0, RSTR, zrow, 0)
        pltpu.sync_copy(zb_v, acc_sh.at[pl.ds(sid * RSTR, RSTR)])

        pltpu.sync_copy(xu_hbm, xu_v)
        pltpu.sync_copy(cb_hbm, cb_v)
        pltpu.sync_copy(src_hbm.at[wid], src_v)
        pltpu.sync_copy(dst_hbm.at[wid], dst_v)

        if with_count:
            one0 = jnp.where(lax.iota(jnp.int32, L) == 0, 1.0, 0.0).astype(jnp.float32)

            def onerow(i, _):
                msg_v[i, pl.ds(L, L)] = one0
                return 0

            lax.fori_loop(0, KB, onerow, 0)

        plsc.subcore_barrier()

        for j in range(NBUF):
            pltpu.async_copy(xw_hbm.at[src_v.at[j]], rows_v.at[j], sems[j])

        @pl.loop(0, nblk, step=NBUF)
        def _round(b0):
            for j in range(NBUF):
                b = b0 + j
                pltpu.make_async_copy(
                    xw_hbm.at[src_v.at[b]], rows_v.at[j], sems[j]).wait()
                for g in range(KB // L):
                    s16 = src_v[b, pl.ds(g * L, L)] * HEADS
                    d16 = dst_v[b, pl.ds(g * L, L)] * HEADS
                    zs = []
                    for h in range(HEADS):
                        zh = (plsc.load_gather(xu_v, [s16 + h])
                              - plsc.load_gather(xu_v, [d16 + h])
                              + cb_v[h, :])
                        zs.append(zh)
                    m = jnp.maximum(jnp.maximum(zs[0], zs[1]),
                                    jnp.maximum(zs[2], zs[3]))
                    es = [jnp.exp(z - m) for z in zs]
                    rinv = 1.0 / (es[0] + es[1] + es[2] + es[3])
                    qs = [e * rinv for e in es]
                    for e in range(L):
                        r = g * L + e
                        acc = qs[0][e] * rows_v[j, r, pl.ds(0, L)]
                        for h in range(1, HEADS):
                            acc = acc + qs[h][e] * rows_v[j, r, pl.ds(h * L, L)]
                        msg_v[r, pl.ds(0, L)] = acc
                nb = b + NBUF

                @pl.when(nb < nblk)
                def _prefetch():
                    pltpu.async_copy(
                        xw_hbm.at[src_v.at[nb]], rows_v.at[j], sems[j])

                pltpu.sync_copy(msg_v, acc_sh.at[dst_v.at[b]], add=True)

        plsc.subcore_barrier()
        pltpu.sync_copy(acc_sh.at[pl.ds(sid * RSTR, RSTR)],
                        out_hbm.at[cid, pl.ds(sid * RSTR, RSTR)])

    return body


@functools.cache
def _edge_b(nblk, cout):
    mesh = plsc.VectorSubcoreMesh(
        core_axis_name="c", subcore_axis_name="s", num_cores=NC, num_subcores=NS)

    @functools.partial(
        pl.kernel,
        out_type=jax.ShapeDtypeStruct((NC, NPAD, cout), jnp.float32),
        mesh=mesh,
        scratch_types=[
            pltpu.VMEM((nblk, KB), jnp.int32),
            pltpu.VMEM((nblk, KB), jnp.int32),
            pltpu.VMEM((NBUF, KB, cout), jnp.float32),
            pltpu.VMEM((RSTR, cout), jnp.float32),
            pltpu.VMEM_SHARED((NPAD, cout), jnp.float32),
            pltpu.SemaphoreType.DMA,
            pltpu.SemaphoreType.DMA,
            pltpu.SemaphoreType.DMA,
        ],
        compiler_params=_SC_PARAMS,
    )
    def body(src_hbm, dst_hbm, xw_hbm, out_hbm,
             src_v, dst_v, rows_v, zb_v, acc_sh, sem0, sem1, sem2):
        sems = (sem0, sem1, sem2)
        cid = lax.axis_index("c")
        sid = lax.axis_index("s")
        wid = cid * NS + sid

        zero16 = jnp.zeros((L,), jnp.float32)

        def zrow(i, _):
            for j in range(cout // L):
                zb_v[i, pl.ds(j * L, L)] = zero16
            return 0

        lax.fori_loop(0, RSTR, zrow, 0)
        pltpu.sync_copy(zb_v, acc_sh.at[pl.ds(sid * RSTR, RSTR)])
        pltpu.sync_copy(src_hbm.at[wid], src_v)
        pltpu.sync_copy(dst_hbm.at[wid], dst_v)

        plsc.subcore_barrier()

        for j in range(NBUF):
            pltpu.async_copy(xw_hbm.at[src_v.at[j]], rows_v.at[j], sems[j])

        @pl.loop(0, nblk, step=NBUF)
        def _round(b0):
            for j in range(NBUF):
                b = b0 + j
                pltpu.make_async_copy(
                    xw_hbm.at[src_v.at[b]], rows_v.at[j], sems[j]).wait()
                pltpu.sync_copy(rows_v.at[j], acc_sh.at[dst_v.at[b]], add=True)
                nb = b + NBUF

                @pl.when(nb < nblk)
                def _prefetch():
                    pltpu.async_copy(
                        xw_hbm.at[src_v.at[nb]], rows_v.at[j], sems[j])

        plsc.subcore_barrier()
        pltpu.sync_copy(acc_sh.at[pl.ds(sid * RSTR, RSTR)],
                        out_hbm.at[cid, pl.ds(sid * RSTR, RSTR)])

    return body


def _mm(a, b):
    return jnp.dot(a, b, preferred_element_type=jnp.float32)


def _prep1_body(x_ref, w_ref, u_ref, xw_ref, xu_ref):
    x = x_ref[...]
    xw_ref[...] = _mm(x, w_ref[...])
    xu_ref[...] = _mm(x, u_ref[...])


def _prep2_body(acc_ref, b_ref, w_ref, u_ref, x1_ref, xw_ref, xu_ref, inv_ref):
    cnt = acc_ref[0, :N, 16:17] + acc_ref[1, :N, 16:17]
    inv = 1.0 / cnt
    inv_ref[...] = inv
    xl = jax.nn.relu((acc_ref[0, :N, 0:16] + acc_ref[1, :N, 0:16]) * inv + b_ref[...])
    x1_ref[...] = xl
    xw_ref[...] = _mm(xl, w_ref[...])
    xu_ref[...] = _mm(xl, u_ref[...])


def _prep_a_body(acc_ref, inv_ref, b_ref, w_ref, u_ref, xw_ref, xu_ref):
    xl = jax.nn.relu((acc_ref[0, :N, :] + acc_ref[1, :N, :]) * inv_ref[...] + b_ref[...])
    xw_ref[...] = _mm(xl, w_ref[...])
    xu_ref[...] = _mm(xl, u_ref[...])


def _prep_b4_body(acc_ref, inv_ref, b_ref, w_ref, x2_ref, xw_ref):
    xl = jax.nn.relu((acc_ref[0, :N, :] + acc_ref[1, :N, :]) * inv_ref[...] + b_ref[...])
    x2_ref[...] = xl
    xw_ref[...] = _mm(xl, w_ref[...])


def _prep_b_body(acc_ref, inv_ref, b_ref, w_ref, xw_ref):
    xl = jax.nn.relu((acc_ref[0, :N, :] + acc_ref[1, :N, :]) * inv_ref[...] + b_ref[...])
    xw_ref[...] = _mm(xl, w_ref[...])


def _head_body(x1_ref, x2_ref, acc_ref, inv_ref, b6_ref,
               l1w_ref, l1b_ref, l2w_ref, l2b_ref, low_ref, lob_ref, out_ref):
    x3 = (acc_ref[0, :N, :] + acc_ref[1, :N, :]) * inv_ref[...] + b6_ref[...]
    x4 = (_mm(x1_ref[...], l1w_ref[0:16, :])
          + _mm(x2_ref[...], l1w_ref[16:32, :])
          + _mm(jax.nn.relu(x3), l1w_ref[32:96, :])
          + l1b_ref[...])
    x5 = _mm(jax.nn.relu(x4), l2w_ref[...]) + l2b_ref[...]
    x6 = _mm(jax.nn.relu(x5), low_ref[...]) + lob_ref[...]
    out_ref[...] = 1.0 / (1.0 + jnp.exp(-x6))


def _tc(body, out_shapes, *args):
    return pl.pallas_call(body, out_shape=out_shapes)(*args)


def kernel(x, edge_index, W1, U1, C1, B1, W2, U2, C2, B2, W3, U3, C3, B3,
           W4, U4, C4, B4, W5, U5, C5, B5, W6, U6, C6, B6,
           L1W, L1B, L2W, L2B, LOW, LOB):
    e_raw = edge_index.shape[1]
    e_tot = e_raw + N
    nblk = _ceil_div(_ceil_div(e_tot, NW * KB), NBUF) * NBUF
    ep = NW * nblk * KB
    pad = ep - e_tot

    loop_idx = jnp.arange(N, dtype=jnp.int32)
    src = jnp.concatenate([
        edge_index[0].astype(jnp.int32), loop_idx,
        jnp.zeros((pad,), jnp.int32)]).reshape(NW, nblk, KB)
    dst = jnp.concatenate([
        edge_index[1].astype(jnp.int32), loop_idx,
        jnp.full((pad,), N, jnp.int32)]).reshape(NW, nblk, KB)

    f32 = jnp.float32
    sd = jax.ShapeDtypeStruct

    cb1 = jnp.broadcast_to(C1[:, None], (HEADS, L))
    cb2 = jnp.broadcast_to(C2[:, None], (HEADS, L))
    cb3 = jnp.broadcast_to(C3[:, None], (HEADS, L))

    xw1, xu1 = _tc(_prep1_body, (sd((N, 64), f32), sd((N, HEADS), f32)),
                   x, W1, U1)
    acc1 = _edge_a(nblk, True)(src, dst, xw1, xu1.reshape(-1), cb1)

    x1, xw2, xu2, inv = _tc(
        _prep2_body,
        (sd((N, 16), f32), sd((N, 64), f32), sd((N, HEADS), f32), sd((N, 1), f32)),
        acc1, B1, W2, U2)
    acc2 = _edge_a(nblk, False)(src, dst, xw2, xu2.reshape(-1), cb2)

    xw3, xu3 = _tc(_prep_a_body, (sd((N, 64), f32), sd((N, HEADS), f32)),
                   acc2, inv, B2, W3, U3)
    acc3 = _edge_a(nblk, False)(src, dst, xw3, xu3.reshape(-1), cb3)

    x2, xw4 = _tc(_prep_b4_body, (sd((N, 16), f32), sd((N, 16), f32)),
                  acc3, inv, B3, W4)
    acc4 = _edge_b(nblk, 16)(src, dst, xw4)

    xw5 = _tc(_prep_b_body, sd((N, 32), f32), acc4, inv, B4, W5)
    acc5 = _edge_b(nblk, 32)(src, dst, xw5)

    xw6 = _tc(_prep_b_body, sd((N, 64), f32), acc5, inv, B5, W6)
    acc6 = _edge_b(nblk, 64)(src, dst, xw6)

    out = _tc(_head_body, sd((N, 1), f32),
              x1, x2, acc6, inv, B6, L1W, L1B, L2W, L2B, LOW, LOB)
    return out


# speedup vs baseline: 21.6392x; 1.0173x over previous
import functools

import jax
import jax.numpy as jnp
from jax import lax
from jax.experimental import pallas as pl
from jax.experimental.pallas import tpu as pltpu
from jax.experimental.pallas import tpu_sc as plsc

N = 10000
HEADS = 4
NC, NS, L = 2, 16, 16
NW = NC * NS
KB = 128
NPAD = 10240
RSTR = NPAD // NS
NBUF = 3
ZROWS = 64

_SC_PARAMS = pltpu.CompilerParams(
    needs_layout_passes=False, use_tc_tiling_on_sc=False)


def _ceil_div(a, b):
    return -(-a // b)


@functools.cache
def _edge_a(nblk, with_count):
    cols = 32 if with_count else 16
    mesh = plsc.VectorSubcoreMesh(
        core_axis_name="c", subcore_axis_name="s", num_cores=NC, num_subcores=NS)

    @functools.partial(
        pl.kernel,
        out_type=jax.ShapeDtypeStruct((NC, NPAD, cols), jnp.float32),
        mesh=mesh,
        scratch_types=[
            pltpu.VMEM((nblk, KB), jnp.int32),
            pltpu.VMEM((nblk, KB), jnp.int32),
            pltpu.VMEM((N * HEADS,), jnp.float32),
            pltpu.VMEM((HEADS, L), jnp.float32),
            pltpu.VMEM((NBUF, KB, 64), jnp.float32),
            pltpu.VMEM((NBUF, KB, cols), jnp.float32),
            pltpu.VMEM((ZROWS, cols), jnp.float32),
            pltpu.VMEM_SHARED((NPAD, cols), jnp.float32),
            pltpu.SemaphoreType.DMA,
            pltpu.SemaphoreType.DMA,
            pltpu.SemaphoreType.DMA,
            pltpu.SemaphoreType.DMA,
            pltpu.SemaphoreType.DMA,
            pltpu.SemaphoreType.DMA,
        ],
        compiler_params=_SC_PARAMS,
    )
    def body(src_hbm, dst_hbm, xw_hbm, xu_hbm, cb_hbm, out_hbm,
             src_v, dst_v, xu_v, cb_v, rows_v, msg_v, zb_v, acc_sh,
             sem0, sem1, sem2, sem3, sem4, sem5):
        sems = (sem0, sem1, sem2)
        ssems = (sem3, sem4, sem5)
        cid = lax.axis_index("c")
        sid = lax.axis_index("s")
        wid = cid * NS + sid

        zero16 = jnp.zeros((L,), jnp.float32)

        def zrow(i, _):
            for j in range(cols // L):
                zb_v[i, pl.ds(j * L, L)] = zero16
            return 0

        lax.fori_loop(0, ZROWS, zrow, 0)

        def zchunk(k, _):
            pltpu.sync_copy(
                zb_v, acc_sh.at[pl.ds(sid * RSTR + k * ZROWS, ZROWS)])
            return 0

        lax.fori_loop(0, RSTR // ZROWS, zchunk, 0)

        pltpu.sync_copy(xu_hbm, xu_v)
        pltpu.sync_copy(cb_hbm, cb_v)
        pltpu.sync_copy(src_hbm.at[wid], src_v)
        pltpu.sync_copy(dst_hbm.at[wid], dst_v)

        if with_count:
            one0 = jnp.where(lax.iota(jnp.int32, L) == 0, 1.0, 0.0).astype(jnp.float32)

            def onerow(i, _):
                for jj in range(NBUF):
                    msg_v[jj, i, pl.ds(L, L)] = one0
                return 0

            lax.fori_loop(0, KB, onerow, 0)

        plsc.subcore_barrier()

        for j in range(NBUF):
            pltpu.async_copy(xw_hbm.at[src_v.at[j]], rows_v.at[j], sems[j])

        @pl.loop(0, nblk, step=NBUF)
        def _round(b0):
            for j in range(NBUF):
                b = b0 + j
                pltpu.make_async_copy(
                    xw_hbm.at[src_v.at[b]], rows_v.at[j], sems[j]).wait()

                @pl.when(b >= NBUF)
                def _drain_prev_scatter():
                    pltpu.make_async_copy(
                        msg_v.at[j], acc_sh.at[dst_v.at[b - NBUF]],
                        ssems[j]).wait()

                for g in range(KB // L):
                    s16 = src_v[b, pl.ds(g * L, L)] * HEADS
                    d16 = dst_v[b, pl.ds(g * L, L)] * HEADS
                    zs = []
                    for h in range(HEADS):
                        zh = (plsc.load_gather(xu_v, [s16 + h])
                              - plsc.load_gather(xu_v, [d16 + h])
                              + cb_v[h, :])
                        zs.append(zh)
                    m = jnp.maximum(jnp.maximum(zs[0], zs[1]),
                                    jnp.maximum(zs[2], zs[3]))
                    es = [jnp.exp(z - m) for z in zs]
                    rinv = 1.0 / (es[0] + es[1] + es[2] + es[3])
                    qs = [e * rinv for e in es]
                    for e in range(L):
                        r = g * L + e
                        acc = qs[0][e] * rows_v[j, r, pl.ds(0, L)]
                        for h in range(1, HEADS):
                            acc = acc + qs[h][e] * rows_v[j, r, pl.ds(h * L, L)]
                        msg_v[j, r, pl.ds(0, L)] = acc
                nb = b + NBUF

                @pl.when(nb < nblk)
                def _prefetch():
                    pltpu.async_copy(
                        xw_hbm.at[src_v.at[nb]], rows_v.at[j], sems[j])

                pltpu.async_copy(
                    msg_v.at[j], acc_sh.at[dst_v.at[b]], ssems[j], add=True)

        for j in range(NBUF):
            pltpu.make_async_copy(
                msg_v.at[j], acc_sh.at[dst_v.at[nblk - NBUF + j]],
                ssems[j]).wait()

        plsc.subcore_barrier()
        pltpu.sync_copy(acc_sh.at[pl.ds(sid * RSTR, RSTR)],
                        out_hbm.at[cid, pl.ds(sid * RSTR, RSTR)])

    return body


@functools.cache
def _edge_b(nblk, cout):
    mesh = plsc.VectorSubcoreMesh(
        core_axis_name="c", subcore_axis_name="s", num_cores=NC, num_subcores=NS)

    @functools.partial(
        pl.kernel,
        out_type=jax.ShapeDtypeStruct((NC, NPAD, cout), jnp.float32),
        mesh=mesh,
        scratch_types=[
            pltpu.VMEM((nblk, KB), jnp.int32),
            pltpu.VMEM((nblk, KB), jnp.int32),
            pltpu.VMEM((NBUF, KB, cout), jnp.float32),
            pltpu.VMEM((RSTR, cout), jnp.float32),
            pltpu.VMEM_SHARED((NPAD, cout), jnp.float32),
            pltpu.SemaphoreType.DMA,
            pltpu.SemaphoreType.DMA,
            pltpu.SemaphoreType.DMA,
        ],
        compiler_params=_SC_PARAMS,
    )
    def body(src_hbm, dst_hbm, xw_hbm, out_hbm,
             src_v, dst_v, rows_v, zb_v, acc_sh, sem0, sem1, sem2):
        sems = (sem0, sem1, sem2)
        cid = lax.axis_index("c")
        sid = lax.axis_index("s")
        wid = cid * NS + sid

        zero16 = jnp.zeros((L,), jnp.float32)

        def zrow(i, _):
            for j in range(cout // L):
                zb_v[i, pl.ds(j * L, L)] = zero16
            return 0

        lax.fori_loop(0, RSTR, zrow, 0)
        pltpu.sync_copy(zb_v, acc_sh.at[pl.ds(sid * RSTR, RSTR)])
        pltpu.sync_copy(src_hbm.at[wid], src_v)
        pltpu.sync_copy(dst_hbm.at[wid], dst_v)

        plsc.subcore_barrier()

        for j in range(NBUF):
            pltpu.async_copy(xw_hbm.at[src_v.at[j]], rows_v.at[j], sems[j])

        @pl.loop(0, nblk, step=NBUF)
        def _round(b0):
            for j in range(NBUF):
                b = b0 + j
                pltpu.make_async_copy(
                    xw_hbm.at[src_v.at[b]], rows_v.at[j], sems[j]).wait()
                pltpu.sync_copy(rows_v.at[j], acc_sh.at[dst_v.at[b]], add=True)
                nb = b + NBUF

                @pl.when(nb < nblk)
                def _prefetch():
                    pltpu.async_copy(
                        xw_hbm.at[src_v.at[nb]], rows_v.at[j], sems[j])

        plsc.subcore_barrier()
        pltpu.sync_copy(acc_sh.at[pl.ds(sid * RSTR, RSTR)],
                        out_hbm.at[cid, pl.ds(sid * RSTR, RSTR)])

    return body


def _mm(a, b):
    return jnp.dot(a, b, preferred_element_type=jnp.float32)


def _prep1_body(x_ref, w_ref, u_ref, xw_ref, xu_ref):
    x = x_ref[...]
    xw_ref[...] = _mm(x, w_ref[...])
    xu_ref[...] = _mm(x, u_ref[...])


def _prep2_body(acc_ref, b_ref, w_ref, u_ref, x1_ref, xw_ref, xu_ref, inv_ref):
    cnt = acc_ref[0, :N, 16:17] + acc_ref[1, :N, 16:17]
    inv = 1.0 / cnt
    inv_ref[...] = inv
    xl = jax.nn.relu((acc_ref[0, :N, 0:16] + acc_ref[1, :N, 0:16]) * inv + b_ref[...])
    x1_ref[...] = xl
    xw_ref[...] = _mm(xl, w_ref[...])
    xu_ref[...] = _mm(xl, u_ref[...])


def _prep_a_body(acc_ref, inv_ref, b_ref, w_ref, u_ref, xw_ref, xu_ref):
    xl = jax.nn.relu((acc_ref[0, :N, :] + acc_ref[1, :N, :]) * inv_ref[...] + b_ref[...])
    xw_ref[...] = _mm(xl, w_ref[...])
    xu_ref[...] = _mm(xl, u_ref[...])


def _prep_b4_body(acc_ref, inv_ref, b_ref, w_ref, x2_ref, xw_ref):
    xl = jax.nn.relu((acc_ref[0, :N, :] + acc_ref[1, :N, :]) * inv_ref[...] + b_ref[...])
    x2_ref[...] = xl
    xw_ref[...] = _mm(xl, w_ref[...])


def _prep_b_body(acc_ref, inv_ref, b_ref, w_ref, xw_ref):
    xl = jax.nn.relu((acc_ref[0, :N, :] + acc_ref[1, :N, :]) * inv_ref[...] + b_ref[...])
    xw_ref[...] = _mm(xl, w_ref[...])


def _head_body(x1_ref, x2_ref, acc_ref, inv_ref, b6_ref,
               l1w_ref, l1b_ref, l2w_ref, l2b_ref, low_ref, lob_ref, out_ref):
    x3 = (acc_ref[0, :N, :] + acc_ref[1, :N, :]) * inv_ref[...] + b6_ref[...]
    x4 = (_mm(x1_ref[...], l1w_ref[0:16, :])
          + _mm(x2_ref[...], l1w_ref[16:32, :])
          + _mm(jax.nn.relu(x3), l1w_ref[32:96, :])
          + l1b_ref[...])
    x5 = _mm(jax.nn.relu(x4), l2w_ref[...]) + l2b_ref[...]
    x6 = _mm(jax.nn.relu(x5), low_ref[...]) + lob_ref[...]
    out_ref[...] = 1.0 / (1.0 + jnp.exp(-x6))


def _tc(body, out_shapes, *args):
    return pl.pallas_call(body, out_shape=out_shapes)(*args)


def kernel(x, edge_index, W1, U1, C1, B1, W2, U2, C2, B2, W3, U3, C3, B3,
           W4, U4, C4, B4, W5, U5, C5, B5, W6, U6, C6, B6,
           L1W, L1B, L2W, L2B, LOW, LOB):
    e_raw = edge_index.shape[1]
    e_tot = e_raw + N
    nblk = _ceil_div(_ceil_div(e_tot, NW * KB), NBUF) * NBUF
    ep = NW * nblk * KB
    pad = ep - e_tot

    loop_idx = jnp.arange(N, dtype=jnp.int32)
    src = jnp.concatenate([
        edge_index[0].astype(jnp.int32), loop_idx,
        jnp.zeros((pad,), jnp.int32)]).reshape(NW, nblk, KB)
    dst = jnp.concatenate([
        edge_index[1].astype(jnp.int32), loop_idx,
        jnp.full((pad,), N, jnp.int32)]).reshape(NW, nblk, KB)

    f32 = jnp.float32
    sd = jax.ShapeDtypeStruct

    cb1 = jnp.broadcast_to(C1[:, None], (HEADS, L))
    cb2 = jnp.broadcast_to(C2[:, None], (HEADS, L))
    cb3 = jnp.broadcast_to(C3[:, None], (HEADS, L))

    xw1, xu1 = _tc(_prep1_body, (sd((N, 64), f32), sd((N, HEADS), f32)),
                   x, W1, U1)
    acc1 = _edge_a(nblk, True)(src, dst, xw1, xu1.reshape(-1), cb1)

    x1, xw2, xu2, inv = _tc(
        _prep2_body,
        (sd((N, 16), f32), sd((N, 64), f32), sd((N, HEADS), f32), sd((N, 1), f32)),
        acc1, B1, W2, U2)
    acc2 = _edge_a(nblk, False)(src, dst, xw2, xu2.reshape(-1), cb2)

    xw3, xu3 = _tc(_prep_a_body, (sd((N, 64), f32), sd((N, HEADS), f32)),
                   acc2, inv, B2, W3, U3)
    acc3 = _edge_a(nblk, False)(src, dst, xw3, xu3.reshape(-1), cb3)

    x2, xw4 = _tc(_prep_b4_body, (sd((N, 16), f32), sd((N, 16), f32)),
                  acc3, inv, B3, W4)
    acc4 = _edge_b(nblk, 16)(src, dst, xw4)

    xw5 = _tc(_prep_b_body, sd((N, 32), f32), acc4, inv, B4, W5)
    acc5 = _edge_b(nblk, 32)(src, dst, xw5)

    xw6 = _tc(_prep_b_body, sd((N, 64), f32), acc5, inv, B5, W6)
    acc6 = _edge_b(nblk, 64)(src, dst, xw6)

    out = _tc(_head_body, sd((N, 1), f32),
              x1, x2, acc6, inv, B6, L1W, L1B, L2W, L2B, LOW, LOB)
    return out
